# baseline (speedup 1.0000x reference)
import jax
import jax.numpy as jnp
from jax.experimental import pallas as pl
from jax.experimental.pallas import tpu as pltpu

EPS = 1e-07
B, C, H, W = 16, 4, 512, 512
NSTREAM = 4
PERBLK = 2
STEPS = (B * C) // (NSTREAM * PERBLK)


def _dice_kernel(*refs):
    dom_ref = refs[0]
    pair_refs = refs[1:1 + 2 * NSTREAM]
    out_ref = refs[1 + 2 * NSTREAM]
    acc_ref = refs[2 + 2 * NSTREAM]
    i = pl.program_id(0)
    n = pl.num_programs(0)

    @pl.when(i == 0)
    def _init():
        acc_ref[...] = jnp.zeros_like(acc_ref)

    row = jax.lax.broadcasted_iota(jnp.int32, (B, C), 0)
    col = jax.lax.broadcasted_iota(jnp.int32, (B, C), 1)
    acc_i = jnp.zeros((B, C), jnp.float32)
    acc_c = jnp.zeros((B, C), jnp.float32)
    for q in range(NSTREAM):
        for p in range(PERBLK):
            xq = pair_refs[2 * q][p]
            tq = pair_refs[2 * q + 1][p]
            slab = (i + q * STEPS) * PERBLK + p
            hot = (row == slab // C) & (col == slab % C)
            acc_i += jnp.where(hot, jnp.sum(xq * tq), 0.0)
            acc_c += jnp.where(hot, jnp.sum(xq + tq), 0.0)
    acc_ref[0] += acc_i
    acc_ref[1] += acc_c

    @pl.when(i == n - 1)
    def _epilogue():
        inter = acc_ref[0]
        card = acc_ref[1]
        d0 = dom_ref[:, 0:1]
        d1 = dom_ref[:, 1:2]
        w1 = (d1 > d0).astype(jnp.float32)
        w0 = 1.0 - w1
        i0 = jnp.sum(inter * w0, axis=0, keepdims=True)
        c0 = jnp.sum(card * w0, axis=0, keepdims=True)
        i1 = jnp.sum(inter * w1, axis=0, keepdims=True)
        c1 = jnp.sum(card * w1, axis=0, keepdims=True)
        loss0 = 1.0 - jnp.mean(2.0 * i0 / (c0 + EPS))
        loss1 = 1.0 - jnp.mean(2.0 * i1 / (c1 + EPS))
        lane = jax.lax.broadcasted_iota(jnp.int32, (1, 4), 1)
        out_ref[...] = jnp.where(
            lane == 0, loss0 + loss1, jnp.where(lane == 1, loss0, loss1)
        )


def kernel(x, label_true, domains):
    xr = x.reshape(B * C, H, W)
    tr = label_true.reshape(B * C, H, W)
    specs = [pl.BlockSpec((B, 2), lambda i: (0, 0))]
    operands = [domains]
    for q in range(NSTREAM):
        specs.append(
            pl.BlockSpec((PERBLK, H, W), lambda i, q=q: (i + q * STEPS, 0, 0)))
        specs.append(
            pl.BlockSpec((PERBLK, H, W), lambda i, q=q: (i + q * STEPS, 0, 0)))
        operands.append(xr)
        operands.append(tr)
    out = pl.pallas_call(
        _dice_kernel,
        grid=(STEPS,),
        in_specs=specs,
        out_specs=pl.BlockSpec((1, 4), lambda i: (0, 0)),
        out_shape=jax.ShapeDtypeStruct((1, 4), jnp.float32),
        scratch_shapes=[pltpu.VMEM((2, B, C), jnp.float32)],
    )(*operands)
    return (out[0, 0], (out[0, 1], out[0, 2]))


# speedup vs baseline: 1.0099x; 1.0099x over previous
import jax
import jax.numpy as jnp
from jax.experimental import pallas as pl
from jax.experimental.pallas import tpu as pltpu

EPS = 1e-07
B, C, H, W = 16, 4, 512, 512
NSTREAM = 8
STEPS = (B * C) // NSTREAM


def _dice_kernel(*refs):
    dom_ref = refs[0]
    pair_refs = refs[1:1 + 2 * NSTREAM]
    out_ref = refs[1 + 2 * NSTREAM]
    acc_ref = refs[2 + 2 * NSTREAM]
    i = pl.program_id(0)
    n = pl.num_programs(0)

    @pl.when(i == 0)
    def _init():
        acc_ref[...] = jnp.zeros_like(acc_ref)

    row = jax.lax.broadcasted_iota(jnp.int32, (B, C), 0)
    col = jax.lax.broadcasted_iota(jnp.int32, (B, C), 1)
    acc_i = jnp.zeros((B, C), jnp.float32)
    acc_c = jnp.zeros((B, C), jnp.float32)
    for q in range(NSTREAM):
        xq = pair_refs[2 * q][0]
        tq = pair_refs[2 * q + 1][0]
        slab = i + q * STEPS
        hot = (row == slab // C) & (col == slab % C)
        acc_i += jnp.where(hot, jnp.sum(xq * tq), 0.0)
        acc_c += jnp.where(hot, jnp.sum(xq + tq), 0.0)
    acc_ref[0] += acc_i
    acc_ref[1] += acc_c

    @pl.when(i == n - 1)
    def _epilogue():
        inter = acc_ref[0]
        card = acc_ref[1]
        d0 = dom_ref[:, 0:1]
        d1 = dom_ref[:, 1:2]
        w1 = (d1 > d0).astype(jnp.float32)
        w0 = 1.0 - w1
        i0 = jnp.sum(inter * w0, axis=0, keepdims=True)
        c0 = jnp.sum(card * w0, axis=0, keepdims=True)
        i1 = jnp.sum(inter * w1, axis=0, keepdims=True)
        c1 = jnp.sum(card * w1, axis=0, keepdims=True)
        loss0 = 1.0 - jnp.mean(2.0 * i0 / (c0 + EPS))
        loss1 = 1.0 - jnp.mean(2.0 * i1 / (c1 + EPS))
        lane = jax.lax.broadcasted_iota(jnp.int32, (1, 4), 1)
        out_ref[...] = jnp.where(
            lane == 0, loss0 + loss1, jnp.where(lane == 1, loss0, loss1)
        )


def kernel(x, label_true, domains):
    xr = x.reshape(B * C, H, W)
    tr = label_true.reshape(B * C, H, W)
    specs = [pl.BlockSpec((B, 2), lambda i: (0, 0))]
    operands = [domains]
    for q in range(NSTREAM):
        specs.append(pl.BlockSpec((1, H, W), lambda i, q=q: (i + q * STEPS, 0, 0)))
        specs.append(pl.BlockSpec((1, H, W), lambda i, q=q: (i + q * STEPS, 0, 0)))
        operands.append(xr)
        operands.append(tr)
    out = pl.pallas_call(
        _dice_kernel,
        grid=(STEPS,),
        in_specs=specs,
        out_specs=pl.BlockSpec((1, 4), lambda i: (0, 0)),
        out_shape=jax.ShapeDtypeStruct((1, 4), jnp.float32),
        scratch_shapes=[pltpu.VMEM((2, B, C), jnp.float32)],
    )(*operands)
    return (out[0, 0], (out[0, 1], out[0, 2]))


# speedup vs baseline: 1.0295x; 1.0194x over previous
import jax
import jax.numpy as jnp
from jax.experimental import pallas as pl
from jax.experimental.pallas import tpu as pltpu

EPS = 1e-07
B, C, H, W = 16, 4, 512, 512
NSTREAM = 8
STEPS = (B * C) // NSTREAM


def _dice_kernel(*refs):
    dom_ref = refs[0]
    pair_refs = refs[1:1 + 2 * NSTREAM]
    out_ref = refs[1 + 2 * NSTREAM]
    acc_ref = refs[2 + 2 * NSTREAM]
    i = pl.program_id(0)
    n = pl.num_programs(0)

    @pl.when(i == 0)
    def _init():
        acc_ref[...] = jnp.zeros_like(acc_ref)

    row = jax.lax.broadcasted_iota(jnp.int32, (B, C), 0)
    col = jax.lax.broadcasted_iota(jnp.int32, (B, C), 1)
    acc_i = jnp.zeros((B, C), jnp.float32)
    acc_c = jnp.zeros((B, C), jnp.float32)
    for q in range(NSTREAM):
        xq = pair_refs[2 * q][0]
        tq = pair_refs[2 * q + 1][0]
        slab = i * NSTREAM + q
        hot = (row == slab // C) & (col == slab % C)
        acc_i += jnp.where(hot, jnp.sum(xq * tq), 0.0)
        acc_c += jnp.where(hot, jnp.sum(xq + tq), 0.0)
    acc_ref[0] += acc_i
    acc_ref[1] += acc_c

    @pl.when(i == n - 1)
    def _epilogue():
        inter = acc_ref[0]
        card = acc_ref[1]
        d0 = dom_ref[:, 0:1]
        d1 = dom_ref[:, 1:2]
        w1 = (d1 > d0).astype(jnp.float32)
        w0 = 1.0 - w1
        i0 = jnp.sum(inter * w0, axis=0, keepdims=True)
        c0 = jnp.sum(card * w0, axis=0, keepdims=True)
        i1 = jnp.sum(inter * w1, axis=0, keepdims=True)
        c1 = jnp.sum(card * w1, axis=0, keepdims=True)
        loss0 = 1.0 - jnp.mean(2.0 * i0 / (c0 + EPS))
        loss1 = 1.0 - jnp.mean(2.0 * i1 / (c1 + EPS))
        lane = jax.lax.broadcasted_iota(jnp.int32, (1, 4), 1)
        out_ref[...] = jnp.where(
            lane == 0, loss0 + loss1, jnp.where(lane == 1, loss0, loss1)
        )


def kernel(x, label_true, domains):
    xr = x.reshape(B * C, H, W)
    tr = label_true.reshape(B * C, H, W)
    specs = [pl.BlockSpec((B, 2), lambda i: (0, 0))]
    operands = [domains]
    for q in range(NSTREAM):
        specs.append(
            pl.BlockSpec((1, H, W), lambda i, q=q: (i * NSTREAM + q, 0, 0)))
        specs.append(
            pl.BlockSpec((1, H, W), lambda i, q=q: (i * NSTREAM + q, 0, 0)))
        operands.append(xr)
        operands.append(tr)
    out = pl.pallas_call(
        _dice_kernel,
        grid=(STEPS,),
        in_specs=specs,
        out_specs=pl.BlockSpec((1, 4), lambda i: (0, 0)),
        out_shape=jax.ShapeDtypeStruct((1, 4), jnp.float32),
        scratch_shapes=[pltpu.VMEM((2, B, C), jnp.float32)],
    )(*operands)
    return (out[0, 0], (out[0, 1], out[0, 2]))
